# baseline (speedup 1.0000x reference)
import jax
import jax.numpy as jnp
import numpy as np
from jax.experimental import pallas as pl
from jax.experimental.pallas import tpu as pltpu

N = 512
V = 4
C = 3
F = 64
H = 64
BR = 16
NB = N // BR
N2 = N // 2
E2 = BR * N2
D = V * C
A = 2 * D + D + 2 * D + BR
L = 2 * D + BR
INV_NEIGH = 1.0 / (N - 1)
INV_SQRT_NEIGH = 1.0 / float(np.sqrt(N - 1))


def _silu_h(t):
    return t + t * jnp.tanh(t)


def _sigmoid_h(t):
    return 0.5 + 0.5 * jnp.tanh(t)


def _dup(a):
    return jnp.concatenate([a, a], axis=-1)


def _body(f_ref, f2_ref, xflat_ref, x2_ref, xr_ref, eye_ref,
          w1s_ref, w1s2_ref, w1r_ref, be1_ref, gg_ref, gg2_ref,
          w1v24_ref, we22_ref, be22_ref, wx12_ref, bx12_ref,
          wx22_ref, bx22_ref, wxo24_ref, bxo24_ref,
          winfw_ref, binfw_ref, gaug_ref,
          we2h_ref, be2h_ref, winfh_ref, binfh_ref,
          wh1a_ref, wh1b_ref, bh1_ref, wh2_ref, bh2_ref, who_ref, bho_ref,
          vec_out_ref, fo_ref,
          fs_s, fr_s, n24_s, xta_s, fst_s, mi_s):
    i = pl.program_id(0)

    @pl.when(i == 0)
    def _prologue():
        f = f_ref[...]
        x = xflat_ref[...]
        x2 = x2_ref[...]
        fs_s[...] = jnp.dot(f, w1s_ref[...], preferred_element_type=jnp.float32) + be1_ref[...]
        fr_s[...] = jnp.dot(f, w1r_ref[...], preferred_element_type=jnp.float32)
        n24_s[...] = jnp.dot(x * x, gg_ref[...], preferred_element_type=jnp.float32)
        n24p = jnp.dot(x2 * x2, gg2_ref[...], preferred_element_type=jnp.float32)
        base = jnp.concatenate(
            [x2, jnp.ones((N2, D), jnp.float32), n24p,
             jnp.ones((N2, BR), jnp.float32)], axis=1)
        xta_s[...] = jnp.broadcast_to(base[None], (BR, N2, A)).reshape(E2, A)
        fs2 = jnp.dot(f2_ref[...], w1s2_ref[...],
                      preferred_element_type=jnp.float32) + _dup(be1_ref[...])
        fst_s[...] = jnp.broadcast_to(fs2[None], (BR, N2, 2 * H)).reshape(E2, 2 * H)

    XTA = xta_s[...]
    xr = xr_ref[...]
    n24r = n24_s[pl.ds(i * BR, BR), :]
    fr_blk = fr_s[pl.ds(i * BR, BR), :]

    xra = jnp.concatenate(
        [_dup(xr), n24r, jnp.ones((BR, 2 * D), jnp.float32), eye_ref[...]],
        axis=1)
    prod = (XTA.reshape(BR, N2, A) * xra[:, None, :]).reshape(E2, A)

    la = jnp.dot(prod, gaug_ref[...], preferred_element_type=jnp.float32)
    la = jnp.maximum(la, 1e-30)
    invden = 1.0 / (1.0 + la * jax.lax.rsqrt(la))

    w1a = jnp.concatenate([w1v24_ref[...], _dup(fr_blk)], axis=0)
    h = _silu_h(jnp.dot(la, w1a, preferred_element_type=jnp.float32) + fst_s[...])
    m = _silu_h(jnp.dot(h, we22_ref[...], preferred_element_type=jnp.float32) + be22_ref[...])

    px = _silu_h(jnp.dot(m.astype(jnp.bfloat16), wx12_ref[...],
                         preferred_element_type=jnp.float32) + bx12_ref[...])
    px = _silu_h(jnp.dot(px.astype(jnp.bfloat16), wx22_ref[...],
                         preferred_element_type=jnp.float32) + bx22_ref[...])
    po = jnp.dot(px.astype(jnp.bfloat16), wxo24_ref[...],
                 preferred_element_type=jnp.float32) + bxo24_ref[...]
    w24 = po * invden[:, :2 * D]

    c24 = (w24 * XTA[:, :2 * D]).reshape(BR, N2, 2 * D).sum(axis=1)
    s24 = w24.reshape(BR, N2, 2 * D).sum(axis=1)
    contrib = c24[:, :D] + c24[:, D:]
    wsum = s24[:, :D] + s24[:, D:]
    vec_out_ref[...] = xr + (wsum * xr - contrib) * INV_NEIGH

    tg = jnp.dot(m, winfw_ref[...], preferred_element_type=jnp.float32) + binfw_ref[...]
    mg = m + m * jnp.tanh(tg)
    mi2 = mg.reshape(BR, N2, 2 * H).sum(axis=1)
    mi_s[pl.ds(i * BR, BR), :] = (mi2[:, :H] + mi2[:, H:]) * (0.5 * INV_SQRT_NEIGH)

    @pl.when(i == NB - 1)
    def _epilogue():
        f = f_ref[...]
        md = _silu_h(fs_s[...] + fr_s[...])
        md = _silu_h(jnp.dot(md, we2h_ref[...], preferred_element_type=jnp.float32)
                     + be2h_ref[...])
        ed = _sigmoid_h(jnp.dot(md, winfh_ref[...], preferred_element_type=jnp.float32)
                        + binfh_ref[...])
        mi = mi_s[...] - md * ed * INV_SQRT_NEIGH
        ph = _silu_h(jnp.dot(mi, wh1a_ref[...], preferred_element_type=jnp.float32)
                     + jnp.dot(f, wh1b_ref[...], preferred_element_type=jnp.float32)
                     + bh1_ref[...])
        ph = _silu_h(jnp.dot(ph, wh2_ref[...], preferred_element_type=jnp.float32)
                     + bh2_ref[...])
        fo_ref[...] = jnp.dot(ph, who_ref[...], preferred_element_type=jnp.float32) \
            + bho_ref[...] + f


def _group_sum_matrix():
    g = np.zeros((D, V), dtype=np.float32)
    for v in range(V):
        g[v * C:(v + 1) * C, v] = 1.0
    return g


def _bdiag(a, b):
    za = jnp.zeros((a.shape[0], b.shape[1]), jnp.float32)
    zb = jnp.zeros((b.shape[0], a.shape[1]), jnp.float32)
    return jnp.concatenate(
        [jnp.concatenate([a, za], axis=1), jnp.concatenate([zb, b], axis=1)], axis=0)


_G_NP = _group_sum_matrix()
_GG_NP = _G_NP @ _G_NP.T


def _gaug_np():
    g = np.zeros((A, L), dtype=np.float32)
    g[0:D, 0:D] = -2.0 * _GG_NP
    g[D:2 * D, D:2 * D] = -2.0 * _GG_NP
    g[2 * D:3 * D, 0:D] = np.eye(D)
    g[2 * D:3 * D, D:2 * D] = np.eye(D)
    g[3 * D:5 * D, 0:2 * D] = np.eye(2 * D)
    g[5 * D:A, 2 * D:L] = np.eye(BR)
    return g


_GAUG_NP = _gaug_np()
_EYE_NP = np.eye(BR, dtype=np.float32)


@jax.jit
def kernel(node_vectors, node_features, We1, be1, We2, be2, Wx1, bx1, Wx2, bx2,
           Wxo, bxo, Winf, binf, Wh1, bh1, Wh2, bh2, Who, bho):
    xflat = node_vectors.reshape(N, D)
    G = jnp.asarray(_G_NP)
    GT = G.T
    GG = jnp.asarray(_GG_NP)

    eb = np.zeros((2, 2 * H), dtype=np.float32)
    eb[0, :H] = 1.0
    eb[1, H:] = 1.0
    eb = jnp.asarray(eb)
    w1v12 = (G @ We1[:V]) / C
    wxo12 = Wxo @ GT
    winfw = _bdiag(Winf, Winf) @ eb
    binfw = jnp.tile(binf, 2).reshape(1, 2) @ eb
    w1s = 0.5 * We1[V:V + F]
    w1r = 0.5 * We1[V + F:]

    full = lambda shape: pl.BlockSpec(shape, lambda i: (0, 0))
    blk = lambda shape: pl.BlockSpec(shape, lambda i: (i, 0))

    vec_out, features_out = pl.pallas_call(
        _body,
        grid=(NB,),
        in_specs=[
            full((N, F)),
            full((N2, 2 * F)),
            full((N, D)),
            full((N2, 2 * D)),
            blk((BR, D)),
            full((BR, BR)),
            full((F, H)),
            full((2 * F, 2 * H)),
            full((F, H)),
            full((1, H)),
            full((D, D)),
            full((2 * D, 2 * D)),
            full((2 * D, 2 * H)),
            full((2 * H, 2 * H)),
            full((1, 2 * H)),
            full((2 * H, 2 * H)),
            full((1, 2 * H)),
            full((2 * H, 2 * H)),
            full((1, 2 * H)),
            full((2 * H, 2 * D)),
            full((1, 2 * D)),
            full((2 * H, 2 * H)),
            full((1, 2 * H)),
            full((A, L)),
            full((H, H)),
            full((1, H)),
            full((H, 1)),
            full((1, 1)),
            full((H, H)),
            full((F, H)),
            full((1, H)),
            full((H, H)),
            full((1, H)),
            full((H, F)),
            full((1, F)),
        ],
        out_specs=(
            blk((BR, D)),
            full((N, F)),
        ),
        out_shape=(
            jax.ShapeDtypeStruct((N, D), jnp.float32),
            jax.ShapeDtypeStruct((N, F), jnp.float32),
        ),
        scratch_shapes=[
            pltpu.VMEM((N, H), jnp.float32),
            pltpu.VMEM((N, H), jnp.float32),
            pltpu.VMEM((N, D), jnp.float32),
            pltpu.VMEM((E2, A), jnp.float32),
            pltpu.VMEM((E2, 2 * H), jnp.float32),
            pltpu.VMEM((N, H), jnp.float32),
        ],
    )(node_features, node_features.reshape(N2, 2 * F), xflat,
      xflat.reshape(N2, 2 * D), xflat, jnp.asarray(_EYE_NP),
      w1s, _bdiag(w1s, w1s), w1r, 0.5 * be1.reshape(1, H), GG, _bdiag(GG, GG),
      0.5 * _bdiag(w1v12, w1v12), 0.5 * _bdiag(We2, We2),
      0.5 * jnp.tile(be2, 2).reshape(1, 2 * H),
      (0.5 * _bdiag(Wx1, Wx1)).astype(jnp.bfloat16),
      0.5 * jnp.tile(bx1, 2).reshape(1, 2 * H),
      (0.5 * _bdiag(Wx2, Wx2)).astype(jnp.bfloat16),
      0.5 * jnp.tile(bx2, 2).reshape(1, 2 * H),
      _bdiag(wxo12, wxo12).astype(jnp.bfloat16),
      jnp.tile(bxo.reshape(1, V) @ GT, (1, 2)),
      0.5 * winfw, 0.5 * binfw, jnp.asarray(_GAUG_NP),
      0.5 * We2, 0.5 * be2.reshape(1, H), 0.5 * Winf, 0.5 * binf.reshape(1, 1),
      0.5 * Wh1[:H], 0.5 * Wh1[H:], 0.5 * bh1.reshape(1, H), 0.5 * Wh2,
      0.5 * bh2.reshape(1, H), Who, bho.reshape(1, F))

    return vec_out.reshape(N, V, C), features_out


# speedup vs baseline: 1.0266x; 1.0266x over previous
import jax
import jax.numpy as jnp
import numpy as np
from jax.experimental import pallas as pl
from jax.experimental.pallas import tpu as pltpu

N = 512
V = 4
C = 3
F = 64
H = 64
BR = 16
NB = N // BR
N2 = N // 2
E2 = BR * N2
D = V * C
A = 2 * D + D + 2 * D + BR
L = 2 * D + BR
INV_NEIGH = 1.0 / (N - 1)
INV_SQRT_NEIGH = 1.0 / float(np.sqrt(N - 1))


def _silu_h(t):
    return t + t * jnp.tanh(t)


def _sigmoid_h(t):
    return 0.5 + 0.5 * jnp.tanh(t)


def _dup(a):
    return jnp.concatenate([a, a], axis=-1)


def _body(f_ref, f2_ref, xflat_ref, x2_ref, xr_ref, eye_ref,
          w1s_ref, w1s2_ref, w1r_ref, be1_ref, gg_ref, gg2_ref,
          w1v24_ref, we22_ref, be22_ref, wx12_ref, bx12_ref,
          wx22_ref, bx22_ref, wxo24_ref, bxo24_ref,
          winfw_ref, binfw_ref, gaug_ref,
          we2h_ref, be2h_ref, winfh_ref, binfh_ref,
          wh1a_ref, wh1b_ref, bh1_ref, wh2_ref, bh2_ref, who_ref, bho_ref,
          vec_out_ref, fo_ref,
          fs_s, fr_s, n24_s, xta_s, fst_s, mi_s):
    i = pl.program_id(0)

    @pl.when(i == 0)
    def _prologue():
        f = f_ref[...]
        x = xflat_ref[...]
        x2 = x2_ref[...]
        fs_s[...] = jnp.dot(f, w1s_ref[...], preferred_element_type=jnp.float32) + be1_ref[...]
        fr_s[...] = jnp.dot(f, w1r_ref[...], preferred_element_type=jnp.float32)
        n24_s[...] = jnp.dot(x * x, gg_ref[...], preferred_element_type=jnp.float32)
        n24p = jnp.dot(x2 * x2, gg2_ref[...], preferred_element_type=jnp.float32)
        base = jnp.concatenate(
            [x2, jnp.ones((N2, D), jnp.float32), n24p,
             jnp.ones((N2, BR), jnp.float32)], axis=1)
        xta_s[...] = jnp.broadcast_to(base[None], (BR, N2, A)).reshape(E2, A)
        fs2 = jnp.dot(f2_ref[...], w1s2_ref[...],
                      preferred_element_type=jnp.float32) + _dup(be1_ref[...])
        fst_s[...] = jnp.broadcast_to(fs2[None], (BR, N2, 2 * H)).reshape(E2, 2 * H)

    XTA = xta_s[...]
    xr = xr_ref[...]
    n24r = n24_s[pl.ds(i * BR, BR), :]
    fr_blk = fr_s[pl.ds(i * BR, BR), :]

    xra = jnp.concatenate(
        [_dup(xr), n24r, jnp.ones((BR, 2 * D), jnp.float32), eye_ref[...]],
        axis=1)
    prod = (XTA.reshape(BR, N2, A) * xra[:, None, :]).reshape(E2, A)

    la = jnp.dot(prod, gaug_ref[...], preferred_element_type=jnp.float32)
    la = jnp.maximum(la, 1e-30)
    invden = 1.0 / (1.0 + la * jax.lax.rsqrt(la))

    w1a = jnp.concatenate([w1v24_ref[...], _dup(fr_blk)], axis=0)
    h = _silu_h(jnp.dot(la, w1a, preferred_element_type=jnp.float32) + fst_s[...])
    m = _silu_h(jnp.dot(h, we22_ref[...], preferred_element_type=jnp.float32) + be22_ref[...])

    px = _silu_h(jnp.dot(m, wx12_ref[...], preferred_element_type=jnp.float32) + bx12_ref[...])
    px = _silu_h(jnp.dot(px, wx22_ref[...], preferred_element_type=jnp.float32) + bx22_ref[...])
    po = jnp.dot(px, wxo24_ref[...], preferred_element_type=jnp.float32) + bxo24_ref[...]
    w24 = po * invden[:, :2 * D]

    c24 = (w24 * XTA[:, :2 * D]).reshape(BR, N2, 2 * D).sum(axis=1)
    s24 = w24.reshape(BR, N2, 2 * D).sum(axis=1)
    contrib = c24[:, :D] + c24[:, D:]
    wsum = s24[:, :D] + s24[:, D:]
    vec_out_ref[...] = xr + (wsum * xr - contrib) * INV_NEIGH

    tg = jnp.dot(m, winfw_ref[...], preferred_element_type=jnp.float32) + binfw_ref[...]
    mg = m + m * jnp.tanh(tg)
    mi2 = mg.reshape(BR, N2, 2 * H).sum(axis=1)
    mi_s[pl.ds(i * BR, BR), :] = (mi2[:, :H] + mi2[:, H:]) * (0.5 * INV_SQRT_NEIGH)

    @pl.when(i == NB - 1)
    def _epilogue():
        f = f_ref[...]
        md = _silu_h(fs_s[...] + fr_s[...])
        md = _silu_h(jnp.dot(md, we2h_ref[...], preferred_element_type=jnp.float32)
                     + be2h_ref[...])
        ed = _sigmoid_h(jnp.dot(md, winfh_ref[...], preferred_element_type=jnp.float32)
                        + binfh_ref[...])
        mi = mi_s[...] - md * ed * INV_SQRT_NEIGH
        ph = _silu_h(jnp.dot(mi, wh1a_ref[...], preferred_element_type=jnp.float32)
                     + jnp.dot(f, wh1b_ref[...], preferred_element_type=jnp.float32)
                     + bh1_ref[...])
        ph = _silu_h(jnp.dot(ph, wh2_ref[...], preferred_element_type=jnp.float32)
                     + bh2_ref[...])
        fo_ref[...] = jnp.dot(ph, who_ref[...], preferred_element_type=jnp.float32) \
            + bho_ref[...] + f


def _group_sum_matrix():
    g = np.zeros((D, V), dtype=np.float32)
    for v in range(V):
        g[v * C:(v + 1) * C, v] = 1.0
    return g


def _bdiag(a, b):
    za = jnp.zeros((a.shape[0], b.shape[1]), jnp.float32)
    zb = jnp.zeros((b.shape[0], a.shape[1]), jnp.float32)
    return jnp.concatenate(
        [jnp.concatenate([a, za], axis=1), jnp.concatenate([zb, b], axis=1)], axis=0)


_G_NP = _group_sum_matrix()
_GG_NP = _G_NP @ _G_NP.T


def _gaug_np():
    g = np.zeros((A, L), dtype=np.float32)
    g[0:D, 0:D] = -2.0 * _GG_NP
    g[D:2 * D, D:2 * D] = -2.0 * _GG_NP
    g[2 * D:3 * D, 0:D] = np.eye(D)
    g[2 * D:3 * D, D:2 * D] = np.eye(D)
    g[3 * D:5 * D, 0:2 * D] = np.eye(2 * D)
    g[5 * D:A, 2 * D:L] = np.eye(BR)
    return g


_GAUG_NP = _gaug_np()
_EYE_NP = np.eye(BR, dtype=np.float32)


@jax.jit
def kernel(node_vectors, node_features, We1, be1, We2, be2, Wx1, bx1, Wx2, bx2,
           Wxo, bxo, Winf, binf, Wh1, bh1, Wh2, bh2, Who, bho):
    xflat = node_vectors.reshape(N, D)
    G = jnp.asarray(_G_NP)
    GT = G.T
    GG = jnp.asarray(_GG_NP)

    eb = np.zeros((2, 2 * H), dtype=np.float32)
    eb[0, :H] = 1.0
    eb[1, H:] = 1.0
    eb = jnp.asarray(eb)
    w1v12 = (G @ We1[:V]) / C
    wxo12 = Wxo @ GT
    winfw = _bdiag(Winf, Winf) @ eb
    binfw = jnp.tile(binf, 2).reshape(1, 2) @ eb
    w1s = 0.5 * We1[V:V + F]
    w1r = 0.5 * We1[V + F:]

    full = lambda shape: pl.BlockSpec(shape, lambda i: (0, 0))
    blk = lambda shape: pl.BlockSpec(shape, lambda i: (i, 0))

    vec_out, features_out = pl.pallas_call(
        _body,
        grid=(NB,),
        in_specs=[
            full((N, F)),
            full((N2, 2 * F)),
            full((N, D)),
            full((N2, 2 * D)),
            blk((BR, D)),
            full((BR, BR)),
            full((F, H)),
            full((2 * F, 2 * H)),
            full((F, H)),
            full((1, H)),
            full((D, D)),
            full((2 * D, 2 * D)),
            full((2 * D, 2 * H)),
            full((2 * H, 2 * H)),
            full((1, 2 * H)),
            full((2 * H, 2 * H)),
            full((1, 2 * H)),
            full((2 * H, 2 * H)),
            full((1, 2 * H)),
            full((2 * H, 2 * D)),
            full((1, 2 * D)),
            full((2 * H, 2 * H)),
            full((1, 2 * H)),
            full((A, L)),
            full((H, H)),
            full((1, H)),
            full((H, 1)),
            full((1, 1)),
            full((H, H)),
            full((F, H)),
            full((1, H)),
            full((H, H)),
            full((1, H)),
            full((H, F)),
            full((1, F)),
        ],
        out_specs=(
            blk((BR, D)),
            full((N, F)),
        ),
        out_shape=(
            jax.ShapeDtypeStruct((N, D), jnp.float32),
            jax.ShapeDtypeStruct((N, F), jnp.float32),
        ),
        scratch_shapes=[
            pltpu.VMEM((N, H), jnp.float32),
            pltpu.VMEM((N, H), jnp.float32),
            pltpu.VMEM((N, D), jnp.float32),
            pltpu.VMEM((E2, A), jnp.float32),
            pltpu.VMEM((E2, 2 * H), jnp.float32),
            pltpu.VMEM((N, H), jnp.float32),
        ],
    )(node_features, node_features.reshape(N2, 2 * F), xflat,
      xflat.reshape(N2, 2 * D), xflat, jnp.asarray(_EYE_NP),
      w1s, _bdiag(w1s, w1s), w1r, 0.5 * be1.reshape(1, H), GG, _bdiag(GG, GG),
      0.5 * _bdiag(w1v12, w1v12), 0.5 * _bdiag(We2, We2),
      0.5 * jnp.tile(be2, 2).reshape(1, 2 * H),
      0.5 * _bdiag(Wx1, Wx1), 0.5 * jnp.tile(bx1, 2).reshape(1, 2 * H),
      0.5 * _bdiag(Wx2, Wx2), 0.5 * jnp.tile(bx2, 2).reshape(1, 2 * H),
      _bdiag(wxo12, wxo12), jnp.tile(bxo.reshape(1, V) @ GT, (1, 2)),
      0.5 * winfw, 0.5 * binfw, jnp.asarray(_GAUG_NP),
      0.5 * We2, 0.5 * be2.reshape(1, H), 0.5 * Winf, 0.5 * binf.reshape(1, 1),
      0.5 * Wh1[:H], 0.5 * Wh1[H:], 0.5 * bh1.reshape(1, H), 0.5 * Wh2,
      0.5 * bh2.reshape(1, H), Who, bho.reshape(1, F))

    return vec_out.reshape(N, V, C), features_out
